# Initial kernel scaffold; baseline (speedup 1.0000x reference)
import jax
import jax.numpy as jnp
from jax.experimental import pallas as pl


def kernel(loc_data, conf_data, blur_data, expression_data, illumination_data, occlusion_data, pose_data, priors, targets):
    raise NotImplementedError("write your pallas kernel here")


# speedup vs baseline: 64.8609x; 64.8609x over previous
import jax
import jax.numpy as jnp
from jax import lax
from jax.experimental import pallas as pl
from jax.experimental.pallas import tpu as pltpu

_THRESHOLD = 0.35
_NEGPOS_RATIO = 7
_VAR0, _VAR1 = 0.1, 0.2
_B = 32
_P = 16800
_NOBJ = 32
_ROWS = 8
_COLS = 2176
_PPAD = _ROWS * _COLS


def _smooth_l1(x):
    ax = jnp.abs(x)
    return jnp.where(ax < 1.0, 0.5 * x * x, ax - 0.5)


def _lse2(a, b):
    m = jnp.maximum(a, b)
    return m + jnp.log(jnp.exp(a - m) + jnp.exp(b - m))


def _lse3(a, b, c):
    m = jnp.maximum(jnp.maximum(a, b), c)
    return m + jnp.log(jnp.exp(a - m) + jnp.exp(b - m) + jnp.exp(c - m))


def _mbl_body(tg_ref, priors_ref, loc_ref, conf_ref, blur_ref, expr_ref,
              illu_ref, occl_ref, pose_ref,
              out_l, out_cpos, out_cneg, out_b, out_e, out_i, out_o, out_p,
              out_np):
    f32 = jnp.float32
    bimg = pl.program_id(0)

    pcx = priors_ref[0]
    pcy = priors_ref[1]
    pw = priors_ref[2]
    ph = priors_ref[3]
    pxmin = pcx - pw / 2.0
    pymin = pcy - ph / 2.0
    pxmax = pcx + pw / 2.0
    pymax = pcy + ph / 2.0
    area_b = (pxmax - pxmin) * (pymax - pymin)

    riota = lax.broadcasted_iota(jnp.int32, (_ROWS, _COLS), 0)
    ciota = lax.broadcasted_iota(jnp.int32, (_ROWS, _COLS), 1)
    piota = riota * _COLS + ciota
    valid = piota < _P

    best_ov = jnp.full((_ROWS, _COLS), -1.0, f32)
    best_j = jnp.zeros((_ROWS, _COLS), jnp.int32)
    bp_list = []
    for j in range(_NOBJ):
        x1 = tg_ref[bimg, j, 0]
        y1 = tg_ref[bimg, j, 1]
        x2 = tg_ref[bimg, j, 2]
        y2 = tg_ref[bimg, j, 3]
        iw = jnp.maximum(jnp.minimum(x2, pxmax) - jnp.maximum(x1, pxmin), 0.0)
        ih = jnp.maximum(jnp.minimum(y2, pymax) - jnp.maximum(y1, pymin), 0.0)
        inter = iw * ih
        area_a = (x2 - x1) * (y2 - y1)
        ov = inter / (area_a + area_b - inter)
        ov = jnp.where(valid, ov, -1.0)
        upd = ov > best_ov
        best_ov = jnp.where(upd, ov, best_ov)
        best_j = jnp.where(upd, j, best_j)
        mx = jnp.max(ov)
        bp_list.append(jnp.min(jnp.where(ov == mx, piota, _PPAD)))
    for j in range(_NOBJ):
        m = piota == bp_list[j]
        best_ov = jnp.where(m, 2.0, best_ov)
        best_j = jnp.where(m, j, best_j)

    pos = jnp.logical_and(best_ov >= _THRESHOLD, valid)

    z = jnp.zeros((_ROWS, _COLS), f32)
    mx1 = z; my1 = z; mx2 = z; my2 = z
    ablur = z; aexpr = z; aillu = z; aoccl = z; apose = z
    for j in range(_NOBJ):
        m = best_j == j
        mx1 = jnp.where(m, tg_ref[bimg, j, 0], mx1)
        my1 = jnp.where(m, tg_ref[bimg, j, 1], my1)
        mx2 = jnp.where(m, tg_ref[bimg, j, 2], mx2)
        my2 = jnp.where(m, tg_ref[bimg, j, 3], my2)
        ablur = jnp.where(m, tg_ref[bimg, j, 4], ablur)
        aexpr = jnp.where(m, tg_ref[bimg, j, 5], aexpr)
        aillu = jnp.where(m, tg_ref[bimg, j, 6], aillu)
        aoccl = jnp.where(m, tg_ref[bimg, j, 7], aoccl)
        apose = jnp.where(m, tg_ref[bimg, j, 8], apose)

    g_cx = ((mx1 + mx2) / 2.0 - pcx) / (_VAR0 * pw)
    g_cy = ((my1 + my2) / 2.0 - pcy) / (_VAR0 * ph)
    g_w = jnp.log((mx2 - mx1) / pw) / _VAR1
    g_h = jnp.log((my2 - my1) / ph) / _VAR1
    sl = (_smooth_l1(loc_ref[0, 0] - g_cx) + _smooth_l1(loc_ref[0, 1] - g_cy)
          + _smooth_l1(loc_ref[0, 2] - g_w) + _smooth_l1(loc_ref[0, 3] - g_h))
    loss_l = jnp.sum(jnp.where(pos, sl, 0.0))

    def ce2(ref, att):
        h0 = ref[0, 0]
        h1 = ref[0, 1]
        picked = jnp.where(att == 0.0, h0, h1)
        return _lse2(h0, h1) - picked

    def ce3(ref, att):
        h0 = ref[0, 0]
        h1 = ref[0, 1]
        h2 = ref[0, 2]
        picked = jnp.where(att == 0.0, h0, jnp.where(att == 1.0, h1, h2))
        return _lse3(h0, h1, h2) - picked

    loss_b = jnp.sum(jnp.where(pos, ce3(blur_ref, ablur), 0.0))
    loss_e = jnp.sum(jnp.where(pos, ce2(expr_ref, aexpr), 0.0))
    loss_i = jnp.sum(jnp.where(pos, ce2(illu_ref, aillu), 0.0))
    loss_o = jnp.sum(jnp.where(pos, ce3(occl_ref, aoccl), 0.0))
    loss_p = jnp.sum(jnp.where(pos, ce2(pose_ref, apose), 0.0))

    c0 = conf_ref[0, 0]
    c1 = conf_ref[0, 1]
    lse_c = _lse2(c0, c1)
    ce_pos = lse_c - c1
    ce_neg = lse_c - c0
    loss_c_pos = jnp.sum(jnp.where(pos, ce_pos, 0.0))
    lcp = jnp.where(pos, 0.0, jnp.where(valid, ce_neg, 0.0))

    num_pos_i = jnp.sum(jnp.where(pos, 1, 0))
    k = jnp.minimum(_NEGPOS_RATIO * num_pos_i, _P - 1)

    bits = lax.bitcast_convert_type(lcp, jnp.int32)
    lo = jnp.int32(0)
    hi = jnp.int32(2147483647)
    for _ in range(31):
        mid = lo + lax.div(hi - lo, jnp.int32(2))
        cnt = jnp.sum(jnp.where(bits >= mid, 1, 0))
        ge_k = cnt >= k
        lo = jnp.where(ge_k, mid, lo)
        hi = jnp.where(ge_k, hi, mid)
    t = lax.bitcast_convert_type(lo, f32)
    gt = lcp > t
    cnt_gt = jnp.sum(jnp.where(gt, 1.0, 0.0))
    sum_gt = jnp.sum(jnp.where(gt, lcp, 0.0))
    loss_c_neg = sum_gt + (k.astype(f32) - cnt_gt) * t

    out_l[0, 0, 0] = loss_l
    out_cpos[0, 0, 0] = loss_c_pos
    out_cneg[0, 0, 0] = loss_c_neg
    out_b[0, 0, 0] = loss_b
    out_e[0, 0, 0] = loss_e
    out_i[0, 0, 0] = loss_i
    out_o[0, 0, 0] = loss_o
    out_p[0, 0, 0] = loss_p
    out_np[0, 0, 0] = num_pos_i.astype(f32)


def _prep(x):
    c = x.shape[-1]
    xt = jnp.transpose(x, (0, 2, 1))
    xt = jnp.pad(xt, ((0, 0), (0, 0), (0, _PPAD - _P)))
    return xt.reshape(_B, c, _ROWS, _COLS)


def kernel(loc_data, conf_data, blur_data, expression_data, illumination_data,
           occlusion_data, pose_data, priors, targets):
    locT = _prep(loc_data)
    confT = _prep(conf_data)
    blurT = _prep(blur_data)
    exprT = _prep(expression_data)
    illuT = _prep(illumination_data)
    occlT = _prep(occlusion_data)
    poseT = _prep(pose_data)
    priorsT = jnp.pad(priors.T, ((0, 0), (0, _PPAD - _P))).reshape(
        4, _ROWS, _COLS)

    def head_spec(c):
        return pl.BlockSpec((1, c, _ROWS, _COLS), lambda b: (b, 0, 0, 0))

    outs = pl.pallas_call(
        _mbl_body,
        grid=(_B,),
        in_specs=[
            pl.BlockSpec(memory_space=pltpu.SMEM),
            pl.BlockSpec((4, _ROWS, _COLS), lambda b: (0, 0, 0)),
            head_spec(4), head_spec(2), head_spec(3), head_spec(2),
            head_spec(2), head_spec(3), head_spec(2),
        ],
        out_specs=[pl.BlockSpec((1, 1, 1), lambda b: (b, 0, 0),
                                memory_space=pltpu.SMEM)] * 9,
        out_shape=[jax.ShapeDtypeStruct((_B, 1, 1), jnp.float32)] * 9,
    )(targets, priorsT, locT, confT, blurT, exprT, illuT, occlT, poseT)

    (s_l, s_cpos, s_cneg, s_b, s_e, s_i, s_o, s_p, s_np) = [
        jnp.sum(o) for o in outs]
    n = jnp.maximum(s_np, 1.0)
    loss_c = s_cpos + s_cneg
    return (s_l / n, loss_c / n, s_b / n, s_e / n, s_i / n, s_o / n, s_p / n)
